# Initial kernel scaffold; baseline (speedup 1.0000x reference)
import jax
import jax.numpy as jnp
from jax.experimental import pallas as pl


def kernel(gps_idx, table):
    raise NotImplementedError("write your pallas kernel here")


# speedup vs baseline: 2.9725x; 2.9725x over previous
import functools

import jax
import jax.numpy as jnp
from jax import lax
from jax.experimental import pallas as pl
from jax.experimental.pallas import tpu as pltpu
from jax.experimental.pallas import tpu_sc as plsc

NC = 2
NS = 16
NW = NC * NS

B = 4096 * 50
D = 128
BPW = B // NW
C = 128
NCHUNK = BPW // C


def _make_gather():
    mesh = plsc.VectorSubcoreMesh(core_axis_name="c", subcore_axis_name="s")

    @functools.partial(
        pl.kernel,
        mesh=mesh,
        out_type=jax.ShapeDtypeStruct((B, D), jnp.float32),
        scratch_types=[
            pltpu.VMEM((NCHUNK, C), jnp.int32),
            pltpu.VMEM((C, D), jnp.float32),
            pltpu.SemaphoreType.DMA,
        ],
    )
    def gather_kernel(idx_hbm, table_hbm, out_hbm, idx_v, buf, sem):
        wid = lax.axis_index("s") * NC + lax.axis_index("c")
        base = wid * BPW
        pltpu.sync_copy(idx_hbm.at[wid], idx_v)

        def body(c, carry):
            pltpu.async_copy(table_hbm.at[idx_v.at[c]], buf, sem).wait()
            pltpu.sync_copy(buf, out_hbm.at[pl.ds(base + c * C, C)])
            return carry

        lax.fori_loop(0, NCHUNK, body, 0)

    return gather_kernel


_gather = _make_gather()


def kernel(gps_idx, table):
    idx = gps_idx.astype(jnp.int32).reshape(NW, NCHUNK, C)
    out = _gather(idx, table)
    return out.reshape(gps_idx.shape + (D,))


# speedup vs baseline: 3.3126x; 1.1144x over previous
import functools

import jax
import jax.numpy as jnp
from jax import lax
from jax.experimental import pallas as pl
from jax.experimental.pallas import tpu as pltpu
from jax.experimental.pallas import tpu_sc as plsc

NC = 2
NS = 16
NW = NC * NS

B = 4096 * 50
D = 128
BPW = B // NW
C = 128
NCHUNK = BPW // C
NBUF = 5
NGROUP = NCHUNK // NBUF


def _make_gather():
    mesh = plsc.VectorSubcoreMesh(core_axis_name="c", subcore_axis_name="s")

    @functools.partial(
        pl.kernel,
        mesh=mesh,
        out_type=jax.ShapeDtypeStruct((B, D), jnp.float32),
        scratch_types=[
            pltpu.VMEM((NCHUNK, C), jnp.int32),
        ]
        + [pltpu.VMEM((C, D), jnp.float32) for _ in range(NBUF)]
        + [pltpu.SemaphoreType.DMA for _ in range(2 * NBUF)],
    )
    def gather_kernel(idx_hbm, table_hbm, out_hbm, idx_v, *bufs_and_sems):
        bufs = bufs_and_sems[:NBUF]
        sem_g = bufs_and_sems[NBUF:2 * NBUF]
        sem_s = bufs_and_sems[2 * NBUF:]
        wid = lax.axis_index("s") * NC + lax.axis_index("c")
        base = wid * BPW
        pltpu.sync_copy(idx_hbm.at[wid], idx_v)

        def gather(chunk, j):
            pltpu.async_copy(table_hbm.at[idx_v.at[chunk]], bufs[j], sem_g[j])

        def store(chunk, j):
            pltpu.async_copy(
                bufs[j], out_hbm.at[pl.ds(base + chunk * C, C)], sem_s[j])

        for j in range(NBUF):
            gather(j, j)

        def body(g, carry):
            c0 = g * NBUF
            for j in range(NBUF):
                pltpu.make_async_copy(
                    table_hbm.at[idx_v.at[0]], bufs[j], sem_g[j]).wait()
                store(c0 + j, j)
            @pl.when(g < NGROUP - 1)
            def _():
                for j in range(NBUF):
                    pltpu.make_async_copy(
                        bufs[j], out_hbm.at[pl.ds(base, C)], sem_s[j]).wait()
                    gather(c0 + NBUF + j, j)
            return carry

        lax.fori_loop(0, NGROUP, body, 0)
        for j in range(NBUF):
            pltpu.make_async_copy(
                bufs[j], out_hbm.at[pl.ds(base, C)], sem_s[j]).wait()

    return gather_kernel


_gather = _make_gather()


def kernel(gps_idx, table):
    idx = gps_idx.astype(jnp.int32).reshape(NW, NCHUNK, C)
    out = _gather(idx, table)
    return out.reshape(gps_idx.shape + (D,))


# speedup vs baseline: 5.9313x; 1.7905x over previous
import functools

import jax
import jax.numpy as jnp
from jax import lax
from jax.experimental import pallas as pl
from jax.experimental.pallas import tpu as pltpu
from jax.experimental.pallas import tpu_sc as plsc

NC = 2
NS = 16
NW = NC * NS

R = 4096
K = 50
D = 128
RPW = R // NW
NBUF = 8
NGROUP = RPW // NBUF


def _make_gather():
    mesh = plsc.VectorSubcoreMesh(core_axis_name="c", subcore_axis_name="s")

    @functools.partial(
        pl.kernel,
        mesh=mesh,
        out_type=jax.ShapeDtypeStruct((R, K, D), jnp.float32),
        scratch_types=[
            pltpu.VMEM((RPW, K), jnp.int32),
        ]
        + [pltpu.VMEM((K, D), jnp.float32) for _ in range(NBUF)]
        + [pltpu.SemaphoreType.DMA for _ in range(2 * NBUF)],
    )
    def gather_kernel(idx_hbm, table_hbm, out_hbm, idx_v, *bufs_and_sems):
        bufs = bufs_and_sems[:NBUF]
        sem_g = bufs_and_sems[NBUF:2 * NBUF]
        sem_s = bufs_and_sems[2 * NBUF:]
        wid = lax.axis_index("s") * NC + lax.axis_index("c")
        base = wid * RPW
        pltpu.sync_copy(idx_hbm.at[pl.ds(base, RPW)], idx_v)

        def gather(row, j):
            pltpu.async_copy(table_hbm.at[idx_v.at[row]], bufs[j], sem_g[j])

        def store(row, j):
            pltpu.async_copy(bufs[j], out_hbm.at[base + row], sem_s[j])

        for j in range(NBUF):
            gather(j, j)

        def body(g, carry):
            r0 = g * NBUF
            for j in range(NBUF):
                pltpu.make_async_copy(
                    table_hbm.at[idx_v.at[0]], bufs[j], sem_g[j]).wait()
                store(r0 + j, j)
            @pl.when(g < NGROUP - 1)
            def _():
                for j in range(NBUF):
                    pltpu.make_async_copy(
                        bufs[j], out_hbm.at[base], sem_s[j]).wait()
                    gather(r0 + NBUF + j, j)
            return carry

        lax.fori_loop(0, NGROUP, body, 0)
        for j in range(NBUF):
            pltpu.make_async_copy(
                bufs[j], out_hbm.at[base], sem_s[j]).wait()

    return gather_kernel


_gather = _make_gather()


def kernel(gps_idx, table):
    return _gather(gps_idx.astype(jnp.int32), table)


# speedup vs baseline: 10.4009x; 1.7536x over previous
import functools

import jax
import jax.numpy as jnp
from jax import lax
from jax.experimental import pallas as pl
from jax.experimental.pallas import tpu as pltpu
from jax.experimental.pallas import tpu_sc as plsc

NC = 2
NS = 16
NW = NC * NS

R = 4096
K = 50
D = 128
RPW = R // NW
NBUF = 5
NGROUP = K // NBUF


def _make_gather():
    mesh = plsc.VectorSubcoreMesh(core_axis_name="c", subcore_axis_name="s")

    @functools.partial(
        pl.kernel,
        mesh=mesh,
        out_type=jax.ShapeDtypeStruct((K, R, D), jnp.float32),
        scratch_types=[
            pltpu.VMEM((K, RPW), jnp.int32),
        ]
        + [pltpu.VMEM((RPW, D), jnp.float32) for _ in range(NBUF)]
        + [pltpu.SemaphoreType.DMA for _ in range(2 * NBUF)],
    )
    def gather_kernel(idx_hbm, table_hbm, out_hbm, idx_v, *bufs_and_sems):
        bufs = bufs_and_sems[:NBUF]
        sem_g = bufs_and_sems[NBUF:2 * NBUF]
        sem_s = bufs_and_sems[2 * NBUF:]
        wid = lax.axis_index("s") * NC + lax.axis_index("c")
        r0 = wid * RPW
        pltpu.sync_copy(idx_hbm.at[:, pl.ds(r0, RPW)], idx_v)

        def gather(k, j):
            pltpu.async_copy(table_hbm.at[idx_v.at[k]], bufs[j], sem_g[j])

        def store(k, j):
            pltpu.async_copy(
                bufs[j], out_hbm.at[k, pl.ds(r0, RPW)], sem_s[j])

        for j in range(NBUF):
            gather(j, j)

        def body(g, carry):
            k0 = g * NBUF
            for j in range(NBUF):
                pltpu.make_async_copy(
                    table_hbm.at[idx_v.at[0]], bufs[j], sem_g[j]).wait()
                store(k0 + j, j)
            @pl.when(g < NGROUP - 1)
            def _():
                for j in range(NBUF):
                    pltpu.make_async_copy(
                        bufs[j], out_hbm.at[0, pl.ds(r0, RPW)], sem_s[j]).wait()
                    gather(k0 + NBUF + j, j)
            return carry

        lax.fori_loop(0, NGROUP, body, 0)
        for j in range(NBUF):
            pltpu.make_async_copy(
                bufs[j], out_hbm.at[0, pl.ds(r0, RPW)], sem_s[j]).wait()

    return gather_kernel


_gather = _make_gather()


def kernel(gps_idx, table):
    idx_t = gps_idx.astype(jnp.int32).T
    out_t = _gather(idx_t, table)
    return jnp.transpose(out_t, (1, 0, 2))


# speedup vs baseline: 10.5469x; 1.0140x over previous
import functools

import jax
import jax.numpy as jnp
from jax import lax
from jax.experimental import pallas as pl
from jax.experimental.pallas import tpu as pltpu
from jax.experimental.pallas import tpu_sc as plsc

NC = 2
NS = 16
NW = NC * NS

R = 4096
K = 50
D = 128
RPW = R // NW
HC = 64
NCHUNK = K * (RPW // HC)
NBUF = 10
NGROUP = NCHUNK // NBUF


def _make_gather():
    mesh = plsc.VectorSubcoreMesh(core_axis_name="c", subcore_axis_name="s")

    @functools.partial(
        pl.kernel,
        mesh=mesh,
        out_type=jax.ShapeDtypeStruct((K, R, D), jnp.float32),
        scratch_types=[
            pltpu.VMEM((K, RPW), jnp.int32),
        ]
        + [pltpu.VMEM((HC, D), jnp.float32) for _ in range(NBUF)]
        + [pltpu.SemaphoreType.DMA for _ in range(2 * NBUF)],
    )
    def gather_kernel(idx_hbm, table_hbm, out_hbm, idx_v, *bufs_and_sems):
        bufs = bufs_and_sems[:NBUF]
        sem_g = bufs_and_sems[NBUF:2 * NBUF]
        sem_s = bufs_and_sems[2 * NBUF:]
        wid = lax.axis_index("s") * NC + lax.axis_index("c")
        r0 = wid * RPW
        pltpu.sync_copy(
            idx_hbm.at[pl.ds(0, 16), pl.ds(r0, RPW)],
            idx_v.at[pl.ds(0, 16)])

        def gather(c, j):
            k = c % K
            h = c // K
            pltpu.async_copy(
                table_hbm.at[idx_v.at[k, pl.ds(h * HC, HC)]], bufs[j],
                sem_g[j])

        def store(c, j):
            k = c % K
            h = c // K
            pltpu.async_copy(
                bufs[j], out_hbm.at[k, pl.ds(r0 + h * HC, HC)], sem_s[j])

        for j in range(NBUF):
            gather(j, j)
        pltpu.sync_copy(
            idx_hbm.at[pl.ds(16, K - 16), pl.ds(r0, RPW)],
            idx_v.at[pl.ds(16, K - 16)])

        def body(g, carry):
            c0 = g * NBUF
            for j in range(NBUF):
                pltpu.make_async_copy(
                    table_hbm.at[idx_v.at[0, pl.ds(0, HC)]], bufs[j],
                    sem_g[j]).wait()
                store(c0 + j, j)
            @pl.when(g < NGROUP - 1)
            def _():
                for j in range(NBUF):
                    pltpu.make_async_copy(
                        bufs[j], out_hbm.at[0, pl.ds(r0, HC)], sem_s[j]).wait()
                    gather(c0 + NBUF + j, j)
            return carry

        lax.fori_loop(0, NGROUP, body, 0)
        for j in range(NBUF):
            pltpu.make_async_copy(
                bufs[j], out_hbm.at[0, pl.ds(r0, HC)], sem_s[j]).wait()

    return gather_kernel


_gather = _make_gather()


def kernel(gps_idx, table):
    idx_t = gps_idx.astype(jnp.int32).T
    out_t = _gather(idx_t, table)
    return jnp.transpose(out_t, (1, 0, 2))


# speedup vs baseline: 10.6308x; 1.0080x over previous
import functools

import jax
import jax.numpy as jnp
from jax import lax
from jax.experimental import pallas as pl
from jax.experimental.pallas import tpu as pltpu
from jax.experimental.pallas import tpu_sc as plsc

NC = 2
NS = 16
NW = NC * NS

R = 4096
K = 50
D = 128
RPW = R // NW
HC = 64
NCHUNK = K * (RPW // HC)
NBUF = 10
NGROUP = NCHUNK // NBUF


def _make_gather():
    mesh = plsc.VectorSubcoreMesh(core_axis_name="c", subcore_axis_name="s")

    @functools.partial(
        pl.kernel,
        mesh=mesh,
        out_type=jax.ShapeDtypeStruct((K, R, D), jnp.float32),
        scratch_types=[
            pltpu.VMEM((K, RPW), jnp.int32),
        ]
        + [pltpu.VMEM((HC, D), jnp.float32) for _ in range(NBUF)]
        + [pltpu.SemaphoreType.DMA for _ in range(2 * NBUF)],
    )
    def gather_kernel(idx_hbm, table_hbm, out_hbm, idx_v, *bufs_and_sems):
        bufs = bufs_and_sems[:NBUF]
        sem_g = bufs_and_sems[NBUF:2 * NBUF]
        sem_s = bufs_and_sems[2 * NBUF:]
        wid = lax.axis_index("s") * NC + lax.axis_index("c")
        r0 = wid * RPW
        pltpu.sync_copy(idx_hbm.at[:, pl.ds(r0, RPW)], idx_v)

        def gather(c, j):
            k = c % K
            h = c // K
            pltpu.async_copy(
                table_hbm.at[idx_v.at[k, pl.ds(h * HC, HC)]], bufs[j],
                sem_g[j])

        def store(c, j):
            k = c % K
            h = c // K
            pltpu.async_copy(
                bufs[j], out_hbm.at[k, pl.ds(r0 + h * HC, HC)], sem_s[j])

        for j in range(NBUF):
            gather(j, j)

        def body(g, carry):
            c0 = g * NBUF
            for j in range(NBUF):
                pltpu.make_async_copy(
                    table_hbm.at[idx_v.at[0, pl.ds(0, HC)]], bufs[j],
                    sem_g[j]).wait()
                store(c0 + j, j)
            @pl.when(g < NGROUP - 1)
            def _():
                for j in range(NBUF):
                    pltpu.make_async_copy(
                        bufs[j], out_hbm.at[0, pl.ds(r0, HC)], sem_s[j]).wait()
                    gather(c0 + NBUF + j, j)
            return carry

        lax.fori_loop(0, NGROUP, body, 0)
        for j in range(NBUF):
            pltpu.make_async_copy(
                bufs[j], out_hbm.at[0, pl.ds(r0, HC)], sem_s[j]).wait()

    return gather_kernel


_gather = _make_gather()


def kernel(gps_idx, table):
    idx_t = gps_idx.astype(jnp.int32).T
    out_t = _gather(idx_t, table)
    return jnp.transpose(out_t, (1, 0, 2))
